# Initial kernel scaffold; baseline (speedup 1.0000x reference)
import jax
import jax.numpy as jnp
from jax.experimental import pallas as pl


def kernel(x, ctx_token, params):
    raise NotImplementedError("write your pallas kernel here")


# speedup vs baseline: 9.2859x; 9.2859x over previous
import functools
import jax
import jax.numpy as jnp
from jax import lax
from jax.experimental import pallas as pl
from jax.experimental.pallas import tpu as pltpu

EPSBN = 1e-5
NEG_INF = float('-inf')
F32 = jnp.float32


def _rup(n, m):
    return (n + m - 1) // m * m


def _fps_kernel(posT_ref, q_ref, *, P_real, S_real, S_pad):
    px = posT_ref[0]
    py = posT_ref[1]
    pz = posT_ref[2]
    B, Pp = px.shape
    laneP = lax.broadcasted_iota(jnp.int32, (B, Pp), 1)
    laneS = lax.broadcasted_iota(jnp.int32, (B, S_pad), 1)
    d = (px - px[:, :1]) ** 2 + (py - py[:, :1]) ** 2 + (pz - pz[:, :1]) ** 2
    d = jnp.where(laneP < P_real, d, -1.0)
    qx = jnp.where(laneS == 0, px[:, :1], 0.0)
    qy = jnp.where(laneS == 0, py[:, :1], 0.0)
    qz = jnp.where(laneS == 0, pz[:, :1], 0.0)

    def step(k, carry):
        d, qx, qy, qz = carry
        vmax = jnp.max(d, axis=1, keepdims=True)
        cand = jnp.where(d == vmax, laneP, Pp)
        amax = jnp.min(cand, axis=1, keepdims=True)
        oh = laneP == amax
        nx = jnp.sum(jnp.where(oh, px, 0.0), axis=1, keepdims=True)
        ny = jnp.sum(jnp.where(oh, py, 0.0), axis=1, keepdims=True)
        nz = jnp.sum(jnp.where(oh, pz, 0.0), axis=1, keepdims=True)
        nd = (px - nx) ** 2 + (py - ny) ** 2 + (pz - nz) ** 2
        d = jnp.where(laneP < P_real, jnp.minimum(d, nd), -1.0)
        sel = laneS == k
        qx = qx + jnp.where(sel, nx, 0.0)
        qy = qy + jnp.where(sel, ny, 0.0)
        qz = qz + jnp.where(sel, nz, 0.0)
        return (d, qx, qy, qz)

    d, qx, qy, qz = lax.fori_loop(1, S_real, step, (d, qx, qy, qz))
    q_ref[0] = qx
    q_ref[1] = qy
    q_ref[2] = qz


def _fps(posT, P_real, S_real, S_pad):
    _, B, Pp = posT.shape
    kfn = functools.partial(_fps_kernel, P_real=P_real, S_real=S_real, S_pad=S_pad)
    return pl.pallas_call(
        kfn,
        out_shape=jax.ShapeDtypeStruct((3, B, S_pad), F32),
    )(posT)


def _sa_kernel(*refs, mode, K, r2, S_real, P_real, C1, C2, C3, CH):
    if mode == "sa1":
        (posT_ref, pos_ref, q_ref, tok_ref,
         MT_ref, WbT_ref, WpT_ref, B1_ref, G1_ref, T1_ref,
         W2T_ref, B2_ref, G2_ref, T2_ref,
         W3T_ref, B3_ref, G3_ref, T3_ref,
         out_ref, edges_ref) = refs
    else:
        (posT_ref, pos_ref, xfeat_ref, q_ref,
         WxT_ref, WpT_ref, B1_ref, G1_ref, T1_ref,
         W2T_ref, B2_ref, G2_ref, T2_ref,
         W3T_ref, B3_ref, G3_ref, T3_ref,
         out_ref, edges_ref) = refs

    Sq = out_ref.shape[1]
    Pc = posT_ref.shape[2]
    laneP = lax.broadcasted_iota(jnp.int32, (Sq, Pc), 1)
    laneK = lax.broadcasted_iota(jnp.int32, (Sq, K), 1)

    d2 = jnp.zeros((Sq, Pc), F32)
    for c in range(3):
        qc = q_ref[0, :, c:c + 1]
        pc_row = posT_ref[0, c:c + 1, :]
        d2 = d2 + (qc - pc_row) ** 2
    d2 = jnp.where(laneP < P_real, d2, float('inf'))

    if mode == "sa1":
        A = jnp.zeros((Pc, C1), F32)
        for c in range(3):
            pcol = pos_ref[0, :, c:c + 1]
            A = A + pcol * MT_ref[c:c + 1, :]
    else:
        A = jnp.dot(xfeat_ref[0], WxT_ref[...], preferred_element_type=F32)
        for c in range(3):
            pcol = pos_ref[0, :, c:c + 1]
            A = A + pcol * WpT_ref[c:c + 1, :]

    cs = jnp.zeros((Sq, C1), F32) + B1_ref[...]
    for c in range(3):
        qc = q_ref[0, :, c:c + 1]
        cs = cs - qc * WpT_ref[c:c + 1, :]
    if mode == "sa1":
        for c in range(3):
            tc = tok_ref[0, :, c:c + 1]
            cs = cs + tc * WbT_ref[c:c + 1, :]

    def step(k, carry):
        d2c, vals = carry
        vmin = jnp.min(d2c, axis=1, keepdims=True)
        cand = jnp.where(d2c == vmin, laneP, Pc)
        amin = jnp.min(cand, axis=1, keepdims=True)
        oh = laneP == amin
        g = jnp.dot(oh.astype(F32), A, preferred_element_type=F32)
        edges_ref[pl.ds(k, 1)] = g[None]
        vals = vals + jnp.where(laneK == k, vmin, 0.0)
        d2c = jnp.where(oh, float('inf'), d2c)
        return (d2c, vals)

    vals0 = jnp.zeros((Sq, K), F32)
    _, vals = lax.fori_loop(0, K, step, (d2, vals0))

    penT = jnp.transpose(jnp.where(vals <= r2, 0.0, NEG_INF).astype(F32))

    run_max = jnp.full((Sq, C3), NEG_INF, F32)
    for t in range(K // CH):
        e = edges_ref[t * CH:(t + 1) * CH]
        h = jax.nn.relu(e + cs[None])
        h = h * G1_ref[...] + T1_ref[...]
        h2 = h.reshape(CH * Sq, C1)
        h2 = jax.nn.relu(jnp.dot(h2, W2T_ref[...], preferred_element_type=F32) + B2_ref[...])
        h2 = h2 * G2_ref[...] + T2_ref[...]
        h3 = jax.nn.relu(jnp.dot(h2, W3T_ref[...], preferred_element_type=F32) + B3_ref[...])
        h3 = h3 * G3_ref[...] + T3_ref[...]
        h3 = h3.reshape(CH, Sq, C3)
        h3 = h3 + penT[t * CH:(t + 1) * CH][:, :, None]
        run_max = jnp.maximum(run_max, jnp.max(h3, axis=0))

    rowS = lax.broadcasted_iota(jnp.int32, (Sq, C3), 0)
    out_ref[...] = jnp.where(rowS < S_real, run_max, 0.0)[None]


def _row(v):
    return v.reshape(1, -1)


def _sa1(posT2, pos, q1, token, layers, S_real, P_real, r):
    B = pos.shape[0]
    Sq = q1.shape[1]
    P = pos.shape[1]
    (W1, b1, g1, bt1), (W2, b2, g2, bt2), (W3, b3, g3, bt3) = layers
    C1, C2, C3 = W1.shape[0], W2.shape[0], W3.shape[0]
    Wa, Wb, Wp = W1[:, :3], W1[:, 3:6], W1[:, 6:9]
    sc1 = 1.0 / jnp.sqrt(1.0 + EPSBN)
    ins = [posT2, pos, q1, token.reshape(B, 1, 3),
           (Wa + Wp).T, Wb.T, Wp.T, _row(b1), (g1 * sc1).reshape(1, 1, C1), bt1.reshape(1, 1, C1),
           W2.T, _row(b2), _row(g2 * sc1), _row(bt2),
           W3.T, _row(b3), _row(g3 * sc1), _row(bt3)]
    specs = [
        pl.BlockSpec((1, 3, P), lambda b: (b, 0, 0)),
        pl.BlockSpec((1, P, 3), lambda b: (b, 0, 0)),
        pl.BlockSpec((1, Sq, 3), lambda b: (b, 0, 0)),
        pl.BlockSpec((1, 1, 3), lambda b: (b, 0, 0)),
    ] + [pl.BlockSpec(w.shape, lambda b, n=w.ndim: (0,) * n) for w in ins[4:]]
    kfn = functools.partial(_sa_kernel, mode="sa1", K=64, r2=r * r,
                            S_real=S_real, P_real=P_real, C1=C1, C2=C2, C3=C3, CH=16)
    return pl.pallas_call(
        kfn,
        grid=(B,),
        in_specs=specs,
        out_specs=pl.BlockSpec((1, Sq, C3), lambda b: (b, 0, 0)),
        out_shape=jax.ShapeDtypeStruct((B, Sq, C3), F32),
        scratch_shapes=[pltpu.VMEM((64, Sq, C1), F32)],
    )(*ins)


def _sa2(posT2, pos, xfeat, q2, layers, S_real, P_real, r):
    B = xfeat.shape[0]
    Sq = q2.shape[1]
    Pc = posT2.shape[2]
    Cin = xfeat.shape[2]
    (W1, b1, g1, bt1), (W2, b2, g2, bt2), (W3, b3, g3, bt3) = layers
    C1, C2, C3 = W1.shape[0], W2.shape[0], W3.shape[0]
    Wx, Wp = W1[:, :Cin], W1[:, Cin:Cin + 3]
    sc1 = 1.0 / jnp.sqrt(1.0 + EPSBN)
    ins = [posT2, pos, xfeat, q2,
           Wx.T, Wp.T, _row(b1), (g1 * sc1).reshape(1, 1, C1), bt1.reshape(1, 1, C1),
           W2.T, _row(b2), _row(g2 * sc1), _row(bt2),
           W3.T, _row(b3), _row(g3 * sc1), _row(bt3)]
    specs = [
        pl.BlockSpec((1, 3, Pc), lambda b: (b, 0, 0)),
        pl.BlockSpec((1, Pc, 3), lambda b: (b, 0, 0)),
        pl.BlockSpec((1, Pc, Cin), lambda b: (b, 0, 0)),
        pl.BlockSpec((1, Sq, 3), lambda b: (b, 0, 0)),
    ] + [pl.BlockSpec(w.shape, lambda b, n=w.ndim: (0,) * n) for w in ins[4:]]
    kfn = functools.partial(_sa_kernel, mode="sa2", K=64, r2=r * r,
                            S_real=S_real, P_real=P_real, C1=C1, C2=C2, C3=C3, CH=16)
    return pl.pallas_call(
        kfn,
        grid=(B,),
        in_specs=specs,
        out_specs=pl.BlockSpec((1, Sq, C3), lambda b: (b, 0, 0)),
        out_shape=jax.ShapeDtypeStruct((B, Sq, C3), F32),
        scratch_shapes=[pltpu.VMEM((64, Sq, C1), F32)],
    )(*ins)


def _final_kernel(x_ref, q_ref, WxT_ref, WpT_ref, B1_ref, G1_ref, T1_ref,
                  W2T_ref, B2_ref, G2_ref, T2_ref,
                  W3T_ref, B3_ref, G3_ref, T3_ref, out_ref, *, S_real):
    x = x_ref[0]
    h = jnp.dot(x, WxT_ref[...], preferred_element_type=F32) + B1_ref[...]
    for c in range(3):
        qc = q_ref[0, :, c:c + 1]
        h = h + qc * WpT_ref[c:c + 1, :]
    h = jax.nn.relu(h) * G1_ref[...] + T1_ref[...]
    h = jax.nn.relu(jnp.dot(h, W2T_ref[...], preferred_element_type=F32) + B2_ref[...])
    h = h * G2_ref[...] + T2_ref[...]
    h = jax.nn.relu(jnp.dot(h, W3T_ref[...], preferred_element_type=F32) + B3_ref[...])
    h = h * G3_ref[...] + T3_ref[...]
    Sq, C3 = h.shape
    rowS = lax.broadcasted_iota(jnp.int32, (Sq, C3), 0)
    h = jnp.where(rowS < S_real, h, NEG_INF)
    out_ref[...] = jnp.max(h, axis=0).reshape(1, 1, C3)


def _final(x2, q2, layers, S_real):
    B, Sq, Cin = x2.shape
    (W1, b1, g1, bt1), (W2, b2, g2, bt2), (W3, b3, g3, bt3) = layers
    C1, C2, C3 = W1.shape[0], W2.shape[0], W3.shape[0]
    Wx, Wp = W1[:, :Cin], W1[:, Cin:Cin + 3]
    sc1 = 1.0 / jnp.sqrt(1.0 + EPSBN)
    ins = [x2, q2,
           Wx.T, Wp.T, _row(b1), _row(g1 * sc1), _row(bt1),
           W2.T, _row(b2), _row(g2 * sc1), _row(bt2),
           W3.T, _row(b3), _row(g3 * sc1), _row(bt3)]
    specs = [
        pl.BlockSpec((1, Sq, Cin), lambda b: (b, 0, 0)),
        pl.BlockSpec((1, Sq, 3), lambda b: (b, 0, 0)),
    ] + [pl.BlockSpec(w.shape, lambda b, n=w.ndim: (0,) * n) for w in ins[2:]]
    kfn = functools.partial(_final_kernel, S_real=S_real)
    out = pl.pallas_call(
        kfn,
        grid=(B,),
        in_specs=specs,
        out_specs=pl.BlockSpec((1, 1, C3), lambda b: (b, 0, 0)),
        out_shape=jax.ShapeDtypeStruct((B, 1, C3), F32),
    )(*ins)
    return out.reshape(B, C3)


def kernel(x, ctx_token, params):
    B, P, _ = x.shape
    S1 = max(1, int(0.2 * P))
    S2 = max(1, int(0.25 * S1))
    S1p = _rup(S1, 8)
    S2p = _rup(S2, 8)

    posT = jnp.transpose(x, (2, 0, 1))
    q1raw = _fps(posT, P, S1, S1p)
    q1 = jnp.transpose(q1raw, (1, 2, 0))

    posT2 = jnp.transpose(x, (0, 2, 1))
    x1 = _sa1(posT2, x, q1, ctx_token, params['sa1'], S1, P, 0.2)

    q2raw = _fps(q1raw, S1, S2, S2p)
    q2 = jnp.transpose(q2raw, (1, 2, 0))

    q1T2 = jnp.transpose(q1, (0, 2, 1))
    x2 = _sa2(q1T2, q1, x1, q2, params['sa2'], S2, S1, 0.4)

    return _final(x2, q2, params['sa3'], S2)
